# baseline (speedup 1.0000x reference)
import jax
import jax.numpy as jnp
from jax.experimental import pallas as pl
from jax.experimental.pallas import tpu as pltpu

NUM_ROWS = 100000
NUM_COLS = 256
N_CHUNKS = 10
ROWS_PER_CHUNK = NUM_ROWS // N_CHUNKS


def _copy_body(x_hbm, o_hbm, sems):
    for i in range(N_CHUNKS):
        pltpu.make_async_copy(
            x_hbm.at[pl.ds(i * ROWS_PER_CHUNK, ROWS_PER_CHUNK), :],
            o_hbm.at[pl.ds(i * ROWS_PER_CHUNK, ROWS_PER_CHUNK), :],
            sems.at[i],
        ).start()
    for i in range(N_CHUNKS):
        pltpu.make_async_copy(
            x_hbm.at[pl.ds(i * ROWS_PER_CHUNK, ROWS_PER_CHUNK), :],
            o_hbm.at[pl.ds(i * ROWS_PER_CHUNK, ROWS_PER_CHUNK), :],
            sems.at[i],
        ).wait()


def kernel(weight):
    return pl.pallas_call(
        _copy_body,
        in_specs=[pl.BlockSpec(memory_space=pl.ANY)],
        out_specs=pl.BlockSpec(memory_space=pl.ANY),
        out_shape=jax.ShapeDtypeStruct((NUM_ROWS, NUM_COLS), jnp.float32),
        scratch_shapes=[pltpu.SemaphoreType.DMA((N_CHUNKS,))],
    )(weight)


# speedup vs baseline: 47.7738x; 47.7738x over previous
import jax
import jax.numpy as jnp
from jax.experimental import pallas as pl
from jax.experimental.pallas import tpu as pltpu

NUM_ROWS = 100000
NUM_COLS = 256
BLOCK_ROWS = 4000


def _copy_body(x_ref, o_ref):
    o_ref[...] = x_ref[...]


def kernel(weight):
    return pl.pallas_call(
        _copy_body,
        grid=(NUM_ROWS // BLOCK_ROWS,),
        in_specs=[pl.BlockSpec((BLOCK_ROWS, NUM_COLS), lambda i: (i, 0))],
        out_specs=pl.BlockSpec((BLOCK_ROWS, NUM_COLS), lambda i: (i, 0)),
        out_shape=jax.ShapeDtypeStruct((NUM_ROWS, NUM_COLS), jnp.float32),
        compiler_params=pltpu.CompilerParams(
            dimension_semantics=("arbitrary",),
        ),
    )(weight)
